# Initial kernel scaffold; baseline (speedup 1.0000x reference)
import jax
import jax.numpy as jnp
from jax.experimental import pallas as pl


def kernel(target_surface_forms, target_priors, input_embeddings, bias):
    raise NotImplementedError("write your pallas kernel here")


# speedup vs baseline: 1.7001x; 1.7001x over previous
import functools

import jax
import jax.numpy as jnp
from jax import lax
from jax.experimental import pallas as pl
from jax.experimental.pallas import tpu as pltpu
from jax.experimental.pallas import tpu_sc as plsc


def _sc_geometry():
    try:
        info = plsc.get_sparse_core_info()
        return info.num_cores, info.num_subcores
    except Exception:
        return 2, 16


@functools.lru_cache(maxsize=None)
def _make_gather(B, V, D):
    NC, NS = _sc_geometry()
    NW = NC * NS
    assert B % NW == 0
    b_per_w = B // NW
    C = 128
    assert b_per_w % C == 0
    n_chunks = b_per_w // C

    mesh = plsc.VectorSubcoreMesh(core_axis_name="c", subcore_axis_name="s")

    @functools.partial(
        pl.kernel,
        out_type=(
            jax.ShapeDtypeStruct((B, D), jnp.float32),
            jax.ShapeDtypeStruct((B,), jnp.float32),
        ),
        mesh=mesh,
        scratch_types=[
            pltpu.VMEM((b_per_w,), jnp.int32),
            pltpu.VMEM((C, D), jnp.float32),
            pltpu.VMEM((b_per_w,), jnp.float32),
            pltpu.SemaphoreType.DMA,
            pltpu.SemaphoreType.DMA,
        ],
    )
    def k(ids_hbm, emb_hbm, bias_hbm, out_emb, out_bias,
          idx_v, rows_v, bias_v, sem_r, sem_b):
        wid = lax.axis_index("s") * NC + lax.axis_index("c")
        base = wid * b_per_w
        pltpu.sync_copy(ids_hbm.at[pl.ds(base, b_per_w)], idx_v)
        for c in range(n_chunks):
            idx_slice = idx_v.at[pl.ds(c * C, C)]
            pltpu.async_copy(emb_hbm.at[idx_slice], rows_v, sem_r)
            pltpu.async_copy(bias_hbm.at[idx_slice],
                             bias_v.at[pl.ds(c * C, C)], sem_b).wait()
            pltpu.make_async_copy(emb_hbm.at[idx_slice], rows_v, sem_r).wait()
            pltpu.sync_copy(rows_v, out_emb.at[pl.ds(base + c * C, C)])
        pltpu.sync_copy(bias_v, out_bias.at[pl.ds(base, b_per_w)])

    return k


def kernel(target_surface_forms, target_priors, input_embeddings, bias):
    B = target_surface_forms.shape[0]
    V, D = input_embeddings.shape
    ids = target_surface_forms[:, 0].astype(jnp.int32)
    gather = _make_gather(B, V, D)
    out_emb, out_bias = gather(ids, input_embeddings, bias.reshape(V))
    return (out_emb, out_bias)


# speedup vs baseline: 1.7474x; 1.0278x over previous
import functools

import jax
import jax.numpy as jnp
from jax import lax
from jax.experimental import pallas as pl
from jax.experimental.pallas import tpu as pltpu
from jax.experimental.pallas import tpu_sc as plsc


def _sc_geometry():
    try:
        info = plsc.get_sparse_core_info()
        return info.num_cores, info.num_subcores
    except Exception:
        return 2, 16


@functools.lru_cache(maxsize=None)
def _make_gather(B, V, D):
    NC, NS = _sc_geometry()
    NW = NC * NS
    assert B % NW == 0
    b_per_w = B // NW
    C = 64
    assert b_per_w % C == 0
    n_chunks = b_per_w // C
    NBUF = 2

    mesh = plsc.VectorSubcoreMesh(core_axis_name="c", subcore_axis_name="s")

    @functools.partial(
        pl.kernel,
        out_type=(
            jax.ShapeDtypeStruct((B, D), jnp.float32),
            jax.ShapeDtypeStruct((B,), jnp.float32),
        ),
        mesh=mesh,
        scratch_types=[
            pltpu.VMEM((b_per_w,), jnp.int32),
            pltpu.VMEM((NBUF, C, D), jnp.float32),
            pltpu.VMEM((b_per_w,), jnp.float32),
            pltpu.SemaphoreType.DMA,
            pltpu.SemaphoreType.DMA,
            pltpu.SemaphoreType.DMA,
        ],
    )
    def k(ids_hbm, emb_hbm, bias_hbm, out_emb, out_bias,
          idx_v, rows_v, bias_v, sem_r, sem_w, sem_b):
        wid = lax.axis_index("s") * NC + lax.axis_index("c")
        base = wid * b_per_w

        pltpu.sync_copy(ids_hbm.at[pl.ds(base, b_per_w)], idx_v)

        def gather(g, buf):
            return pltpu.make_async_copy(
                emb_hbm.at[idx_v.at[pl.ds(g * C, C)]], rows_v.at[buf], sem_r)

        def write(g, buf):
            return pltpu.make_async_copy(
                rows_v.at[buf], out_emb.at[pl.ds(base + g * C, C)], sem_w)

        CB = 128
        for c in range(b_per_w // CB):
            pltpu.async_copy(bias_hbm.at[idx_v.at[pl.ds(c * CB, CB)]],
                             bias_v.at[pl.ds(c * CB, CB)], sem_b)

        gather(0, 0).start()
        if n_chunks > 1:
            gather(1, 1).start()
        for g in range(n_chunks):
            buf = g % NBUF
            gather(g, buf).wait()
            write(g, buf).start()
            if g + NBUF < n_chunks:
                write(g, buf).wait()
                gather(g + NBUF, buf).start()
        for g in range(max(0, n_chunks - NBUF), n_chunks):
            write(g, g % NBUF).wait()

        for c in range(b_per_w // CB):
            pltpu.make_async_copy(bias_hbm.at[idx_v.at[pl.ds(c * CB, CB)]],
                                  bias_v.at[pl.ds(c * CB, CB)], sem_b).wait()
        pltpu.sync_copy(bias_v, out_bias.at[pl.ds(base, b_per_w)])

    return k


def kernel(target_surface_forms, target_priors, input_embeddings, bias):
    B = target_surface_forms.shape[0]
    V, D = input_embeddings.shape
    ids = target_surface_forms[:, 0].astype(jnp.int32)
    gather = _make_gather(B, V, D)
    out_emb, out_bias = gather(ids, input_embeddings, bias.reshape(V))
    return (out_emb, out_bias)
